# baseline (speedup 1.0000x reference)
import functools

import jax
import jax.numpy as jnp
from jax import lax
from jax.experimental import pallas as pl
from jax.experimental.pallas import tpu as pltpu
from jax.experimental.pallas import tpu_sc as plsc

N = 10000
E = 320000
D = 32
STEPS = 10
SLOPE = 0.01

NC = 2
NS = 16
NW = NC * NS
EPW = E // NW
BATCH = 80
OPS = EPW // BATCH
SPB = 5
SUP = OPS // SPB
SROWS = SPB * BATCH
NPAD = 10112
RPT = NPAD // NS

_mesh = plsc.VectorSubcoreMesh(core_axis_name="c", subcore_axis_name="s")


def _worker_id():
    return lax.axis_index("c") * NS + lax.axis_index("s")


@functools.partial(
    pl.kernel,
    mesh=_mesh,
    compiler_params=pltpu.CompilerParams(use_tc_tiling_on_sc=False),
    out_type=jax.ShapeDtypeStruct((E, D), jnp.float32),
    scratch_types=[
        pltpu.VMEM((OPS, BATCH), jnp.int32),
        pltpu.VMEM((OPS, BATCH), jnp.int32),
        pltpu.VMEM((2, SROWS, D), jnp.float32),
        pltpu.VMEM((2, SROWS, D), jnp.float32),
        pltpu.SemaphoreType.DMA,
        pltpu.SemaphoreType.DMA,
    ],
)
def _sc_gather(ps, pd, src3d, dst3d, g1, si, di, ga, gb, semg, semw):
    w = _worker_id()
    pltpu.sync_copy(src3d.at[w], si)
    pltpu.sync_copy(dst3d.at[w], di)
    row0 = w * EPW

    def fire(k, slot):
        for t in range(SPB):
            pltpu.async_copy(
                ps.at[si.at[k * SPB + t]], ga.at[slot, pl.ds(t * BATCH, BATCH)], semg)
            pltpu.async_copy(
                pd.at[di.at[k * SPB + t]], gb.at[slot, pl.ds(t * BATCH, BATCH)], semg)

    fire(0, 0)

    def body(k, carry):
        slot = k % 2
        for t in range(SPB):
            pltpu.make_async_copy(
                ps.at[si.at[k * SPB + t]], ga.at[slot, pl.ds(t * BATCH, BATCH)], semg).wait()
            pltpu.make_async_copy(
                pd.at[di.at[k * SPB + t]], gb.at[slot, pl.ds(t * BATCH, BATCH)], semg).wait()

        @pl.when(k >= 1)
        def _():
            pltpu.make_async_copy(
                ga.at[1 - slot], g1.at[pl.ds(row0 + (k - 1) * SROWS, SROWS)], semw).wait()

        @pl.when(k <= SUP - 2)
        def _():
            fire(k + 1, 1 - slot)

        def addrow(i, c2):
            for rr in range(4):
                for hh in range(2):
                    sl = pl.ds(hh * 16, 16)
                    plsc.addupdate(ga.at[slot, i * 4 + rr, sl],
                                   gb[slot, i * 4 + rr, sl])
            return c2

        lax.fori_loop(0, SROWS // 4, addrow, 0)
        pltpu.async_copy(ga.at[slot], g1.at[pl.ds(row0 + k * SROWS, SROWS)], semw)
        return carry

    lax.fori_loop(0, SUP, body, 0)
    k = SUP - 1
    pltpu.make_async_copy(
        ga.at[k % 2], g1.at[pl.ds(row0 + k * SROWS, SROWS)], semw).wait()


@functools.partial(
    pl.kernel,
    mesh=_mesh,
    compiler_params=pltpu.CompilerParams(use_tc_tiling_on_sc=False),
    out_type=jax.ShapeDtypeStruct((NC, NPAD, D), jnp.float32),
    scratch_types=[
        pltpu.VMEM((OPS, BATCH), jnp.int32),
        pltpu.VMEM((2, SROWS, D), jnp.float32),
        pltpu.VMEM((RPT, D), jnp.float32),
        pltpu.VMEM_SHARED((NPAD, D), jnp.float32),
        pltpu.SemaphoreType.DMA,
    ],
)
def _sc_scatter(e, dst3d, zeros, part, di, eb, zb, acc, seml):
    c = lax.axis_index("c")
    s = lax.axis_index("s")
    w = c * NS + s
    row0 = w * EPW
    pltpu.sync_copy(dst3d.at[w], di)

    def fire(k, slot):
        for t in range(SPB):
            pltpu.async_copy(
                e.at[pl.ds(row0 + (k * SPB + t) * BATCH, BATCH)],
                eb.at[slot, pl.ds(t * BATCH, BATCH)], seml)

    fire(0, 0)
    pltpu.sync_copy(zeros, zb)
    pltpu.sync_copy(zb, acc.at[pl.ds(s * RPT, RPT)])
    plsc.subcore_barrier()

    def body(k, carry):
        slot = k % 2
        for t in range(SPB):
            pltpu.make_async_copy(
                e.at[pl.ds(row0 + (k * SPB + t) * BATCH, BATCH)],
                eb.at[slot, pl.ds(t * BATCH, BATCH)], seml).wait()

        @pl.when(k <= SUP - 2)
        def _():
            fire(k + 1, 1 - slot)

        for t in range(SPB):
            pltpu.sync_copy(
                eb.at[slot, pl.ds(t * BATCH, BATCH)], acc.at[di.at[k * SPB + t]],
                add=True)
        return carry

    lax.fori_loop(0, SUP, body, 0)
    plsc.subcore_barrier()
    pltpu.sync_copy(acc.at[pl.ds(s * RPT, RPT)], zb)
    pltpu.sync_copy(zb, part.at[c, pl.ds(s * RPT, RPT)])


@functools.partial(
    pl.kernel,
    mesh=_mesh,
    compiler_params=pltpu.CompilerParams(use_tc_tiling_on_sc=False),
    out_type=jax.ShapeDtypeStruct((NC, NPAD, D), jnp.float32),
    scratch_types=[
        pltpu.VMEM((OPS, BATCH), jnp.int32),
        pltpu.VMEM((BATCH, D), jnp.float32),
        pltpu.VMEM((RPT, D), jnp.float32),
        pltpu.VMEM_SHARED((NPAD, D), jnp.float32),
    ],
)
def _sc_degree(dst3d, ones, zeros, part, di, ob, zb, acc):
    c = lax.axis_index("c")
    s = lax.axis_index("s")
    w = c * NS + s
    pltpu.sync_copy(dst3d.at[w], di)
    pltpu.sync_copy(ones, ob)
    pltpu.sync_copy(zeros, zb)
    pltpu.sync_copy(zb, acc.at[pl.ds(s * RPT, RPT)])
    plsc.subcore_barrier()

    def body(j, carry):
        pltpu.sync_copy(ob, acc.at[di.at[j]], add=True)
        return carry

    lax.fori_loop(0, OPS, body, 0)
    plsc.subcore_barrier()
    pltpu.sync_copy(acc.at[pl.ds(s * RPT, RPT)], zb)
    pltpu.sync_copy(zb, part.at[c, pl.ds(s * RPT, RPT)])


E4 = E // 4
N4 = N // 4
NPAD4 = NPAD // 4
EPS = 1e-5


def _lrelu(x):
    return jnp.where(x >= 0, x, SLOPE * x)


def _pln(u, m_ref, g, b):
    mu = jnp.dot(u, m_ref[...], preferred_element_type=jnp.float32)
    d = u - mu
    var = jnp.dot(d * d, m_ref[...], preferred_element_type=jnp.float32)
    return d * lax.rsqrt(var + EPS) * g + b


def _enc_nodes_body(x_ref, w_ref, b_ref, ws_ref, wd_ref, h_ref, ps_ref, pd_ref):
    h = _lrelu(jnp.dot(x_ref[...], w_ref[...], preferred_element_type=jnp.float32)
               + b_ref[...])
    h_ref[...] = h
    ps_ref[...] = jnp.dot(h, ws_ref[...], preferred_element_type=jnp.float32)
    pd_ref[...] = jnp.dot(h, wd_ref[...], preferred_element_type=jnp.float32)


def _enc_edges_body(a_ref, w_ref, b_ref, o_ref):
    o_ref[...] = _lrelu(
        jnp.dot(a_ref[...], w_ref[...], preferred_element_type=jnp.float32)
        + b_ref[...])


def _edge_body(e_ref, g_ref, w_ref, m_ref, b_ref, ga_ref, be_ref, o_ref):
    u = (jnp.dot(e_ref[...], w_ref[...], preferred_element_type=jnp.float32)
         + g_ref[...] + b_ref[...])
    o_ref[...] = e_ref[...] + _pln(_lrelu(u), m_ref, ga_ref[...], be_ref[...])


def _node_body(h_ref, p0_ref, p1_ref, c0_ref, c1_ref, wh_ref, wa_ref, m_ref,
               b_ref, ga_ref, be_ref, ws_ref, wd_ref, h2_ref, ps_ref, pd_ref):
    agg = (p0_ref[...] + p1_ref[...]) / jnp.maximum(c0_ref[...] + c1_ref[...], 1.0)
    h = h_ref[...]
    u = (jnp.dot(h, wh_ref[...], preferred_element_type=jnp.float32)
         + jnp.dot(agg, wa_ref[...], preferred_element_type=jnp.float32)
         + b_ref[...])
    h2 = h + _pln(_lrelu(u), m_ref, ga_ref[...], be_ref[...])
    h2_ref[...] = h2
    ps_ref[...] = jnp.dot(h2, ws_ref[...], preferred_element_type=jnp.float32)
    pd_ref[...] = jnp.dot(h2, wd_ref[...], preferred_element_type=jnp.float32)


def _dec_body(h_ref, w1_ref, b1_ref, w2_ref, b2_ref, o_ref):
    z = _lrelu(jnp.dot(h_ref[...], w1_ref[...], preferred_element_type=jnp.float32)
               + b1_ref[...])
    o_ref[...] = (jnp.dot(z, w2_ref[...], preferred_element_type=jnp.float32)
                  + b2_ref[...])


_BE = 8000
_BA = 8000


def _tc_enc_nodes(x4, w4, b4, ws, wd):
    return pl.pallas_call(
        _enc_nodes_body,
        out_shape=[jax.ShapeDtypeStruct((N4, 128), jnp.float32)] * 3,
    )(x4, w4, b4, ws, wd)


def _tc_enc_edges(ea128, w32, b32):
    blk = 2000
    return pl.pallas_call(
        _enc_edges_body,
        grid=(E // 32 // blk,),
        in_specs=[
            pl.BlockSpec((blk, 128), lambda i: (i, 0)),
            pl.BlockSpec((128, 1024), lambda i: (0, 0)),
            pl.BlockSpec((1, 1024), lambda i: (0, 0)),
        ],
        out_specs=pl.BlockSpec((blk, 1024), lambda i: (i, 0)),
        out_shape=jax.ShapeDtypeStruct((E // 32, 1024), jnp.float32),
    )(ea128, w32, b32)


def _tc_edge(e, g1, w, m, b, g, bl):
    full = pl.BlockSpec((128, 128), lambda i: (0, 0))
    vec = pl.BlockSpec((1, 128), lambda i: (0, 0))
    blk = pl.BlockSpec((_BE, 128), lambda i: (i, 0))
    return pl.pallas_call(
        _edge_body,
        grid=(E4 // _BE,),
        in_specs=[blk, blk, full, full, vec, vec, vec],
        out_specs=blk,
        out_shape=jax.ShapeDtypeStruct((E4, 128), jnp.float32),
    )(e, g1, w, m, b, g, bl)


def _tc_node(h, p0, p1, c0, c1, wh, wa, m, b, g, bl, ws, wd):
    return pl.pallas_call(
        _node_body,
        out_shape=[jax.ShapeDtypeStruct((N4, 128), jnp.float32)] * 3,
    )(h, p0, p1, c0, c1, wh, wa, m, b, g, bl, ws, wd)


def _tc_dec(h, w1, b1, w2, b2):
    return pl.pallas_call(
        _dec_body,
        out_shape=jax.ShapeDtypeStruct((N4, 4), jnp.float32),
    )(h, w1, b1, w2, b2)


def kernel(x, edge_index, edge_attr, W_node_enc, b_node_enc, W_edge_enc,
           b_edge_enc, W_edge_mlp, b_edge_mlp, W_node_mlp, b_node_mlp,
           ln_edge_g, ln_edge_b, ln_node_g, ln_node_b,
           W_dec1, b_dec1, W_dec2, b_dec2):
    src3d = edge_index[0].reshape(NW, OPS, BATCH)
    dst3d = edge_index[1].reshape(NW, OPS, BATCH)
    zeros = jnp.zeros((RPT, D), jnp.float32)
    ones = jnp.ones((BATCH, D), jnp.float32)
    eye4 = jnp.eye(4, dtype=jnp.float32)
    blkdiag = lambda w: jnp.kron(eye4, w)
    tile4 = lambda v: jnp.tile(v.reshape(1, D), (1, 4))
    M = jnp.kron(eye4, jnp.full((D, D), 1.0 / D, jnp.float32))
    w_enc_n = jnp.kron(eye4, W_node_enc)
    w_enc_e = jnp.kron(jnp.eye(32, dtype=jnp.float32), W_edge_enc)

    deg = _sc_degree(dst3d, ones, zeros)
    deg4 = deg.reshape(NC, NPAD4, 128)
    d0, d1 = deg4[0, :N4], deg4[1, :N4]

    h, ps, pd = _tc_enc_nodes(
        x.reshape(N4, 512), w_enc_n, tile4(b_node_enc),
        blkdiag(W_edge_mlp[0, D:2 * D]), blkdiag(W_edge_mlp[0, 2 * D:]))
    ps = ps.reshape(N, D)
    pd = pd.reshape(N, D)
    e = _tc_enc_edges(edge_attr.reshape(E // 32, 128), w_enc_e,
                      jnp.tile(b_edge_enc.reshape(1, D), (1, 32)))
    e = e.reshape(E4, 128)

    for t in range(STEPS):
        g1 = _sc_gather(ps, pd, src3d, dst3d)
        e = _tc_edge(e, g1.reshape(E4, 128),
                     blkdiag(W_edge_mlp[t, :D]), M, tile4(b_edge_mlp[t]),
                     tile4(ln_edge_g[t]), tile4(ln_edge_b[t]))
        part = _sc_scatter(e.reshape(E, D), dst3d, zeros)
        part4 = part.reshape(NC, NPAD4, 128)
        tn = min(t + 1, STEPS - 1)
        h, ps, pd = _tc_node(
            h, part4[0, :N4], part4[1, :N4], d0, d1,
            blkdiag(W_node_mlp[t, :D]), blkdiag(W_node_mlp[t, D:]), M,
            tile4(b_node_mlp[t]), tile4(ln_node_g[t]), tile4(ln_node_b[t]),
            blkdiag(W_edge_mlp[tn, D:2 * D]), blkdiag(W_edge_mlp[tn, 2 * D:]))
        ps = ps.reshape(N, D)
        pd = pd.reshape(N, D)

    out = _tc_dec(h, blkdiag(W_dec1), tile4(b_dec1),
                  jnp.kron(eye4, W_dec2), jnp.tile(b_dec2.reshape(1, 1), (1, 4)))
    return out.reshape(N, 1)


# speedup vs baseline: 1.0959x; 1.0959x over previous
import functools

import jax
import jax.numpy as jnp
from jax import lax
from jax.experimental import pallas as pl
from jax.experimental.pallas import tpu as pltpu
from jax.experimental.pallas import tpu_sc as plsc

N = 10000
E = 320000
D = 32
STEPS = 10
SLOPE = 0.01

NC = 2
NS = 16
NW = NC * NS
EPW = E // NW
BATCH = 80
OPS = EPW // BATCH
SPB = 5
SUP = OPS // SPB
SROWS = SPB * BATCH
NPAD = 10112
RPT = NPAD // NS

_mesh = plsc.VectorSubcoreMesh(core_axis_name="c", subcore_axis_name="s")


def _worker_id():
    return lax.axis_index("c") * NS + lax.axis_index("s")


@functools.partial(
    pl.kernel,
    mesh=_mesh,
    compiler_params=pltpu.CompilerParams(use_tc_tiling_on_sc=False),
    out_type=[
        jax.ShapeDtypeStruct((E, D), jnp.float32),
        jax.ShapeDtypeStruct((E, D), jnp.float32),
    ],
    scratch_types=[
        pltpu.VMEM((OPS, BATCH), jnp.int32),
        pltpu.VMEM((OPS, BATCH), jnp.int32),
        pltpu.VMEM((2, SROWS, D), jnp.float32),
        pltpu.VMEM((2, SROWS, D), jnp.float32),
        pltpu.SemaphoreType.DMA,
        pltpu.SemaphoreType.DMA,
    ],
)
def _sc_gather(ps, pd, src3d, dst3d, g1, g2, si, di, ga, gb, semg, semw):
    w = _worker_id()
    pltpu.sync_copy(src3d.at[w], si)
    pltpu.sync_copy(dst3d.at[w], di)
    row0 = w * EPW

    def fire(k, slot):
        for t in range(SPB):
            pltpu.async_copy(
                ps.at[si.at[k * SPB + t]], ga.at[slot, pl.ds(t * BATCH, BATCH)], semg)
            pltpu.async_copy(
                pd.at[di.at[k * SPB + t]], gb.at[slot, pl.ds(t * BATCH, BATCH)], semg)

    fire(0, 0)

    def body(k, carry):
        slot = k % 2
        for t in range(SPB):
            pltpu.make_async_copy(
                ps.at[si.at[k * SPB + t]], ga.at[slot, pl.ds(t * BATCH, BATCH)], semg).wait()
            pltpu.make_async_copy(
                pd.at[di.at[k * SPB + t]], gb.at[slot, pl.ds(t * BATCH, BATCH)], semg).wait()

        @pl.when(k >= 1)
        def _():
            pltpu.make_async_copy(
                ga.at[1 - slot], g1.at[pl.ds(row0 + (k - 1) * SROWS, SROWS)], semw).wait()
            pltpu.make_async_copy(
                gb.at[1 - slot], g2.at[pl.ds(row0 + (k - 1) * SROWS, SROWS)], semw).wait()

        @pl.when(k <= SUP - 2)
        def _():
            fire(k + 1, 1 - slot)

        pltpu.async_copy(ga.at[slot], g1.at[pl.ds(row0 + k * SROWS, SROWS)], semw)
        pltpu.async_copy(gb.at[slot], g2.at[pl.ds(row0 + k * SROWS, SROWS)], semw)
        return carry

    lax.fori_loop(0, SUP, body, 0)
    k = SUP - 1
    pltpu.make_async_copy(
        ga.at[k % 2], g1.at[pl.ds(row0 + k * SROWS, SROWS)], semw).wait()
    pltpu.make_async_copy(
        gb.at[k % 2], g2.at[pl.ds(row0 + k * SROWS, SROWS)], semw).wait()


@functools.partial(
    pl.kernel,
    mesh=_mesh,
    compiler_params=pltpu.CompilerParams(use_tc_tiling_on_sc=False),
    out_type=jax.ShapeDtypeStruct((NC, NPAD, D), jnp.float32),
    scratch_types=[
        pltpu.VMEM((OPS, BATCH), jnp.int32),
        pltpu.VMEM((2, SROWS, D), jnp.float32),
        pltpu.VMEM((RPT, D), jnp.float32),
        pltpu.VMEM_SHARED((NPAD, D), jnp.float32),
        pltpu.SemaphoreType.DMA,
    ],
)
def _sc_scatter(e, dst3d, zeros, part, di, eb, zb, acc, seml):
    c = lax.axis_index("c")
    s = lax.axis_index("s")
    w = c * NS + s
    row0 = w * EPW
    pltpu.sync_copy(dst3d.at[w], di)

    def fire(k, slot):
        for t in range(SPB):
            pltpu.async_copy(
                e.at[pl.ds(row0 + (k * SPB + t) * BATCH, BATCH)],
                eb.at[slot, pl.ds(t * BATCH, BATCH)], seml)

    fire(0, 0)
    pltpu.sync_copy(zeros, zb)
    pltpu.sync_copy(zb, acc.at[pl.ds(s * RPT, RPT)])
    plsc.subcore_barrier()

    def body(k, carry):
        slot = k % 2
        for t in range(SPB):
            pltpu.make_async_copy(
                e.at[pl.ds(row0 + (k * SPB + t) * BATCH, BATCH)],
                eb.at[slot, pl.ds(t * BATCH, BATCH)], seml).wait()

        @pl.when(k <= SUP - 2)
        def _():
            fire(k + 1, 1 - slot)

        for t in range(SPB):
            pltpu.sync_copy(
                eb.at[slot, pl.ds(t * BATCH, BATCH)], acc.at[di.at[k * SPB + t]],
                add=True)
        return carry

    lax.fori_loop(0, SUP, body, 0)
    plsc.subcore_barrier()
    pltpu.sync_copy(acc.at[pl.ds(s * RPT, RPT)], zb)
    pltpu.sync_copy(zb, part.at[c, pl.ds(s * RPT, RPT)])


@functools.partial(
    pl.kernel,
    mesh=_mesh,
    compiler_params=pltpu.CompilerParams(use_tc_tiling_on_sc=False),
    out_type=jax.ShapeDtypeStruct((NC, NPAD, D), jnp.float32),
    scratch_types=[
        pltpu.VMEM((OPS, BATCH), jnp.int32),
        pltpu.VMEM((BATCH, D), jnp.float32),
        pltpu.VMEM((RPT, D), jnp.float32),
        pltpu.VMEM_SHARED((NPAD, D), jnp.float32),
    ],
)
def _sc_degree(dst3d, ones, zeros, part, di, ob, zb, acc):
    c = lax.axis_index("c")
    s = lax.axis_index("s")
    w = c * NS + s
    pltpu.sync_copy(dst3d.at[w], di)
    pltpu.sync_copy(ones, ob)
    pltpu.sync_copy(zeros, zb)
    pltpu.sync_copy(zb, acc.at[pl.ds(s * RPT, RPT)])
    plsc.subcore_barrier()

    def body(j, carry):
        pltpu.sync_copy(ob, acc.at[di.at[j]], add=True)
        return carry

    lax.fori_loop(0, OPS, body, 0)
    plsc.subcore_barrier()
    pltpu.sync_copy(acc.at[pl.ds(s * RPT, RPT)], zb)
    pltpu.sync_copy(zb, part.at[c, pl.ds(s * RPT, RPT)])


E4 = E // 4
N4 = N // 4
NPAD4 = NPAD // 4
EPS = 1e-5


def _lrelu(x):
    return jnp.where(x >= 0, x, SLOPE * x)


def _pln(u, m_ref, g, b):
    mu = jnp.dot(u, m_ref[...], preferred_element_type=jnp.float32)
    d = u - mu
    var = jnp.dot(d * d, m_ref[...], preferred_element_type=jnp.float32)
    return d * lax.rsqrt(var + EPS) * g + b


def _enc_nodes_body(x_ref, w_ref, b_ref, ws_ref, wd_ref, h_ref, ps_ref, pd_ref):
    h = _lrelu(jnp.dot(x_ref[...], w_ref[...], preferred_element_type=jnp.float32)
               + b_ref[...])
    h_ref[...] = h
    ps_ref[...] = jnp.dot(h, ws_ref[...], preferred_element_type=jnp.float32)
    pd_ref[...] = jnp.dot(h, wd_ref[...], preferred_element_type=jnp.float32)


def _enc_edges_body(a_ref, w_ref, b_ref, o_ref):
    o_ref[...] = _lrelu(
        jnp.dot(a_ref[...], w_ref[...], preferred_element_type=jnp.float32)
        + b_ref[...])


def _edge_body(e_ref, g1_ref, g2_ref, w_ref, m_ref, b_ref, ga_ref, be_ref, o_ref):
    u = (jnp.dot(e_ref[...], w_ref[...], preferred_element_type=jnp.float32)
         + g1_ref[...] + g2_ref[...] + b_ref[...])
    o_ref[...] = e_ref[...] + _pln(_lrelu(u), m_ref, ga_ref[...], be_ref[...])


def _node_body(h_ref, p0_ref, p1_ref, c0_ref, c1_ref, wh_ref, wa_ref, m_ref,
               b_ref, ga_ref, be_ref, ws_ref, wd_ref, h2_ref, ps_ref, pd_ref):
    agg = (p0_ref[...] + p1_ref[...]) / jnp.maximum(c0_ref[...] + c1_ref[...], 1.0)
    h = h_ref[...]
    u = (jnp.dot(h, wh_ref[...], preferred_element_type=jnp.float32)
         + jnp.dot(agg, wa_ref[...], preferred_element_type=jnp.float32)
         + b_ref[...])
    h2 = h + _pln(_lrelu(u), m_ref, ga_ref[...], be_ref[...])
    h2_ref[...] = h2
    ps_ref[...] = jnp.dot(h2, ws_ref[...], preferred_element_type=jnp.float32)
    pd_ref[...] = jnp.dot(h2, wd_ref[...], preferred_element_type=jnp.float32)


def _dec_body(h_ref, w1_ref, b1_ref, w2_ref, b2_ref, o_ref):
    z = _lrelu(jnp.dot(h_ref[...], w1_ref[...], preferred_element_type=jnp.float32)
               + b1_ref[...])
    o_ref[...] = (jnp.dot(z, w2_ref[...], preferred_element_type=jnp.float32)
                  + b2_ref[...])


_BE = 8000
_BA = 8000


def _tc_enc_nodes(x4, w4, b4, ws, wd):
    return pl.pallas_call(
        _enc_nodes_body,
        out_shape=[jax.ShapeDtypeStruct((N4, 128), jnp.float32)] * 3,
    )(x4, w4, b4, ws, wd)


def _tc_enc_edges(ea4, w16, b4):
    blk = 8000
    return pl.pallas_call(
        _enc_edges_body,
        grid=(E4 // blk,),
        in_specs=[
            pl.BlockSpec((blk, 16), lambda i: (i, 0)),
            pl.BlockSpec((16, 128), lambda i: (0, 0)),
            pl.BlockSpec((1, 128), lambda i: (0, 0)),
        ],
        out_specs=pl.BlockSpec((blk, 128), lambda i: (i, 0)),
        out_shape=jax.ShapeDtypeStruct((E4, 128), jnp.float32),
    )(ea4, w16, b4)


def _tc_edge(e, g1, g2, w, m, b, g, bl):
    full = pl.BlockSpec((128, 128), lambda i: (0, 0))
    vec = pl.BlockSpec((1, 128), lambda i: (0, 0))
    blk = pl.BlockSpec((_BE, 128), lambda i: (i, 0))
    return pl.pallas_call(
        _edge_body,
        grid=(E4 // _BE,),
        in_specs=[blk, blk, blk, full, full, vec, vec, vec],
        out_specs=blk,
        out_shape=jax.ShapeDtypeStruct((E4, 128), jnp.float32),
    )(e, g1, g2, w, m, b, g, bl)


def _tc_node(h, p0, p1, c0, c1, wh, wa, m, b, g, bl, ws, wd):
    return pl.pallas_call(
        _node_body,
        out_shape=[jax.ShapeDtypeStruct((N4, 128), jnp.float32)] * 3,
    )(h, p0, p1, c0, c1, wh, wa, m, b, g, bl, ws, wd)


def _tc_dec(h, w1, b1, w2, b2):
    return pl.pallas_call(
        _dec_body,
        out_shape=jax.ShapeDtypeStruct((N4, 4), jnp.float32),
    )(h, w1, b1, w2, b2)


def kernel(x, edge_index, edge_attr, W_node_enc, b_node_enc, W_edge_enc,
           b_edge_enc, W_edge_mlp, b_edge_mlp, W_node_mlp, b_node_mlp,
           ln_edge_g, ln_edge_b, ln_node_g, ln_node_b,
           W_dec1, b_dec1, W_dec2, b_dec2):
    src3d = edge_index[0].reshape(NW, OPS, BATCH)
    dst3d = edge_index[1].reshape(NW, OPS, BATCH)
    zeros = jnp.zeros((RPT, D), jnp.float32)
    ones = jnp.ones((BATCH, D), jnp.float32)
    eye4 = jnp.eye(4, dtype=jnp.float32)
    blkdiag = lambda w: jnp.kron(eye4, w)
    tile4 = lambda v: jnp.tile(v.reshape(1, D), (1, 4))
    M = jnp.kron(eye4, jnp.full((D, D), 1.0 / D, jnp.float32))
    w_enc_n = jnp.kron(eye4, W_node_enc)
    w_enc_e = jnp.kron(eye4, W_edge_enc)

    deg = _sc_degree(dst3d, ones, zeros)
    deg4 = deg.reshape(NC, NPAD4, 128)
    d0, d1 = deg4[0, :N4], deg4[1, :N4]

    h, ps, pd = _tc_enc_nodes(
        x.reshape(N4, 512), w_enc_n, tile4(b_node_enc),
        blkdiag(W_edge_mlp[0, D:2 * D]), blkdiag(W_edge_mlp[0, 2 * D:]))
    ps = ps.reshape(N, D)
    pd = pd.reshape(N, D)
    e = _tc_enc_edges(edge_attr.reshape(E4, 16), w_enc_e, tile4(b_edge_enc))

    for t in range(STEPS):
        g1, g2 = _sc_gather(ps, pd, src3d, dst3d)
        e = _tc_edge(e, g1.reshape(E4, 128), g2.reshape(E4, 128),
                     blkdiag(W_edge_mlp[t, :D]), M, tile4(b_edge_mlp[t]),
                     tile4(ln_edge_g[t]), tile4(ln_edge_b[t]))
        part = _sc_scatter(e.reshape(E, D), dst3d, zeros)
        part4 = part.reshape(NC, NPAD4, 128)
        tn = min(t + 1, STEPS - 1)
        h, ps, pd = _tc_node(
            h, part4[0, :N4], part4[1, :N4], d0, d1,
            blkdiag(W_node_mlp[t, :D]), blkdiag(W_node_mlp[t, D:]), M,
            tile4(b_node_mlp[t]), tile4(ln_node_g[t]), tile4(ln_node_b[t]),
            blkdiag(W_edge_mlp[tn, D:2 * D]), blkdiag(W_edge_mlp[tn, 2 * D:]))
        ps = ps.reshape(N, D)
        pd = pd.reshape(N, D)

    out = _tc_dec(h, blkdiag(W_dec1), tile4(b_dec1),
                  jnp.kron(eye4, W_dec2), jnp.tile(b_dec2.reshape(1, 1), (1, 4)))
    return out.reshape(N, 1)
